# baseline (device time: 10296 ns/iter reference)
import jax
import jax.numpy as jnp
from jax import lax
from jax.experimental import pallas as pl
from jax.experimental.pallas import tpu as pltpu

N_DEV = 4
E_PER_DEV = 2


def kernel(x, router_W, route_idx, expert_W):
    n_tok, d_model = x.shape
    d_out = expert_W.shape[2]

    def body(x_ref, rW_ref, idx_ref, eW_ref, out_ref,
             sbuf_ref, rbuf_ref, send_sems, recv_sems):
        my_pos = lax.axis_index("i")
        partner_a = my_pos ^ 1
        partner_b = 3 - my_pos
        diag = my_pos ^ 3

        barrier_sem = pltpu.get_barrier_semaphore()
        for nbr in [partner_a, partner_b, diag]:
            pl.semaphore_signal(
                barrier_sem, inc=1,
                device_id=(nbr,), device_id_type=pl.DeviceIdType.MESH,
            )

        idx = idx_ref[:, :]
        e0 = my_pos * E_PER_DEV
        xm = jnp.concatenate(
            [
                x_ref[:, :] * (idx == e0).astype(jnp.float32),
                x_ref[:, :] * (idx == e0 + 1).astype(jnp.float32),
            ],
            axis=1,
        )
        w_cat = eW_ref[:, :, :].reshape(E_PER_DEV * d_model, d_out)
        partial = jnp.dot(xm, w_cat, preferred_element_type=jnp.float32)
        sbuf_ref[:, :] = partial.astype(jnp.bfloat16)

        pl.semaphore_wait(barrier_sem, 3)

        rdmas = []
        for slot_at_target, target in [(2, diag), (0, partner_a), (1, partner_b)]:
            rdma = pltpu.make_async_remote_copy(
                src_ref=sbuf_ref,
                dst_ref=rbuf_ref.at[slot_at_target],
                send_sem=send_sems.at[slot_at_target],
                recv_sem=recv_sems.at[slot_at_target],
                device_id=(target,),
                device_id_type=pl.DeviceIdType.MESH,
            )
            rdma.start()
            rdmas.append(rdma)

        for rdma in rdmas:
            rdma.wait()

        out_ref[:, :] = partial + (
            rbuf_ref[0].astype(jnp.float32)
            + rbuf_ref[1].astype(jnp.float32)
            + rbuf_ref[2].astype(jnp.float32)
        )

    return pl.pallas_call(
        body,
        out_shape=jax.ShapeDtypeStruct((n_tok, d_out), jnp.float32),
        in_specs=[
            pl.BlockSpec(memory_space=pltpu.VMEM),
            pl.BlockSpec(memory_space=pltpu.VMEM),
            pl.BlockSpec(memory_space=pltpu.VMEM),
            pl.BlockSpec(memory_space=pltpu.VMEM),
        ],
        out_specs=pl.BlockSpec(memory_space=pltpu.VMEM),
        scratch_shapes=[
            pltpu.VMEM((n_tok, d_out), jnp.bfloat16),
            pltpu.VMEM((3, n_tok, d_out), jnp.bfloat16),
            pltpu.SemaphoreType.DMA((3,)),
            pltpu.SemaphoreType.DMA((3,)),
        ],
        compiler_params=pltpu.CompilerParams(collective_id=0),
    )(x, router_W, route_idx, expert_W)
